# baseline (speedup 1.0000x reference)
import functools

import jax
import jax.numpy as jnp
from jax import lax
from jax.experimental import pallas as pl
from jax.experimental.pallas import tpu as pltpu
from jax.experimental.pallas import tpu_sc as plsc

F32 = jnp.float32
I32 = jnp.int32

NC = 2
NS = 16
CH = 256
BN_EPS = 1e-5


GROUPS = 2


@functools.lru_cache(maxsize=None)
def _make_agg(n_nodes, n_pad, h_dim, chunks_pad):
    grp = chunks_pad // GROUPS
    rows_per_tile = n_pad // NS
    mesh = plsc.VectorSubcoreMesh(core_axis_name="c", subcore_axis_name="s")

    def body(h_hbm, src_hbm, dst_hbm, zeros_hbm, out_hbm,
             acc, sidx, didx, rows, *sems):
        c = lax.axis_index("c")
        s = lax.axis_index("s")
        lo = s * rows_per_tile
        pltpu.sync_copy(zeros_hbm.at[pl.ds(lo, rows_per_tile)],
                        acc.at[pl.ds(lo, rows_per_tile)])
        plsc.subcore_barrier()

        for g in range(GROUPS):
            pltpu.sync_copy(src_hbm.at[c, s, g], sidx)
            pltpu.sync_copy(dst_hbm.at[c, s, g], didx)
            @pl.loop(0, grp)
            def _(j):
                ds = pl.ds(j * CH, CH)
                pltpu.async_copy(h_hbm.at[sidx.at[ds]], rows, sems[0]).wait()
                pltpu.sync_copy(rows, acc.at[didx.at[ds]], add=True)

        plsc.subcore_barrier()
        pltpu.sync_copy(acc.at[pl.ds(lo, rows_per_tile)],
                        out_hbm.at[c, pl.ds(lo, rows_per_tile)])

    return pl.kernel(
        body,
        out_type=jax.ShapeDtypeStruct((NC, n_pad, h_dim), F32),
        mesh=mesh,
        scratch_types=[
            pltpu.VMEM_SHARED((n_pad, h_dim), F32),
            pltpu.VMEM((grp * CH,), I32),
            pltpu.VMEM((grp * CH,), I32),
            pltpu.VMEM((CH, h_dim), F32),
            pltpu.SemaphoreType.DMA,
        ],
    )


def _embed_body(ntr, ctr, ptr, tcat, w3, pb, out):
    br = out.shape[0]
    nt = ntr[0]
    ct = ctr[0]
    pt = ptr[0]
    i24 = lax.broadcasted_iota(I32, (br, 24), 1)
    oh = ((i24 == jnp.clip(nt, 0, 7))
          | (i24 == (jnp.clip(ct, 0, 7) + 8))
          | (i24 == (jnp.clip(pt, 0, 7) + 16))).astype(F32)
    t = tcat[...]
    w = w3[...]
    tp = jnp.concatenate(
        [jnp.dot(t[0:8], w[0], preferred_element_type=F32),
         jnp.dot(t[8:16], w[1], preferred_element_type=F32),
         jnp.dot(t[16:24], w[2], preferred_element_type=F32)], axis=0)
    out[...] = jnp.dot(oh, tp, preferred_element_type=F32) + pb[...]


def _layer_body(h_ref, agg_ref, w1, b1, w2, b2, gam, bet, out_ref,
                z2_s, st_s, *, br, n_nodes, residual):
    ph = pl.program_id(0)
    j = pl.program_id(1)

    @pl.when(ph == 0)
    def _():
        @pl.when(j == 0)
        def _():
            st_s[...] = jnp.zeros_like(st_s)
        z = h_ref[...] + agg_ref[0] + agg_ref[1]
        t = jnp.maximum(jnp.dot(z, w1[...], preferred_element_type=F32)
                        + b1[...], 0.0)
        z2 = jnp.dot(t, w2[...], preferred_element_type=F32) + b2[...]
        z2_s[pl.ds(j * br, br), :] = z2
        st_s[0:1, :] += jnp.sum(z2, axis=0, keepdims=True)
        st_s[1:2, :] += jnp.sum(z2 * z2, axis=0, keepdims=True)

    @pl.when(ph == 1)
    def _():
        nf = jnp.float32(n_nodes)
        mu = st_s[0:1, :] / nf
        var = st_s[1:2, :] / nf - mu * mu
        inv = 1.0 / jnp.sqrt(var + BN_EPS)
        z2 = z2_s[pl.ds(j * br, br), :]
        y = (z2 - mu) * inv * gam[...] + bet[...]
        y = jnp.maximum(y, 0.0)
        if residual:
            y = y + h_ref[...]
        out_ref[...] = y


def _pool_body(h_ref, brow_ref, bcol_ref, out_ref, sum_s, max_s, cnt_s,
               *, br, n_graphs):
    j = pl.program_id(0)

    @pl.when(j == 0)
    def _():
        sum_s[...] = jnp.zeros_like(sum_s)
        cnt_s[...] = jnp.zeros_like(cnt_s)
        max_s[...] = jnp.full_like(max_s, -jnp.inf)

    b_row = brow_ref[0]
    b_col = bcol_ref[0]
    h = h_ref[...]
    oht = (lax.broadcasted_iota(I32, (n_graphs, br), 0)
           == b_row).astype(F32)
    sum_s[...] += jnp.dot(oht, h, preferred_element_type=F32)
    cnt_s[...] += jnp.broadcast_to(jnp.sum(oht, axis=1, keepdims=True),
                                   cnt_s.shape)
    neg = jnp.float32(-jnp.inf)
    for g in range(n_graphs):
        m = jnp.max(jnp.where(b_col == g, h, neg), axis=0, keepdims=True)
        max_s[g:g + 1, :] = jnp.maximum(max_s[g:g + 1, :], m)

    cnt = jnp.maximum(cnt_s[...], 1.0)
    out_ref[...] = jnp.concatenate(
        [sum_s[...] / cnt, max_s[...], sum_s[...]], axis=1)


def kernel(x, edge_index, batch, params):
    n = x.shape[0]
    h_dim = params["proj_W"].shape[1]
    n_graphs = 16
    br = 1000
    n_blocks = n // br
    n_pad = ((n + 1 + NS * 8 - 1) // (NS * 8)) * (NS * 8)

    e = edge_index.shape[1]
    chunks_pad = -(-(-(-e // (NC * NS * CH))) // GROUPS) * GROUPS
    e_pad = NC * NS * chunks_pad * CH

    pad_dst = n + (jnp.arange(e_pad - e, dtype=I32) % (n_pad - n))
    src = jnp.concatenate(
        [edge_index[0], jnp.zeros((e_pad - e,), I32)]).reshape(
            NC, NS, GROUPS, (chunks_pad // GROUPS) * CH)
    dst = jnp.concatenate(
        [edge_index[1], pad_dst]).reshape(
            NC, NS, GROUPS, (chunks_pad // GROUPS) * CH)
    zeros_hbm = jnp.zeros((n_pad, h_dim), F32)

    tcat = jnp.zeros((24, h_dim), F32)
    tcat = tcat.at[0:5].set(params["node_emb"])
    tcat = tcat.at[8:14].set(params["comp_emb"])
    tcat = tcat.at[16:22].set(params["pin_emb"])
    w3 = params["proj_W"].reshape(3, h_dim, h_dim)
    pb = params["proj_b"].reshape(1, h_dim)

    xt = x.astype(I32).T.reshape(3, n_blocks, br, 1)
    ntr, ctr, ptr = xt[0], xt[1], xt[2]
    batch_r = batch.astype(I32).reshape(n_blocks, 1, br)
    batch_c = batch.astype(I32).reshape(n_blocks, br, 1)

    idx_spec = pl.BlockSpec((1, br, 1), lambda j: (j, 0, 0))
    full = lambda shp: pl.BlockSpec(shp, lambda j: tuple(0 for _ in shp))
    h = pl.pallas_call(
        _embed_body,
        grid=(n_blocks,),
        in_specs=[idx_spec, idx_spec, idx_spec,
                  full((24, h_dim)), full((3, h_dim, h_dim)),
                  full((1, h_dim))],
        out_specs=pl.BlockSpec((br, h_dim), lambda j: (j, 0)),
        out_shape=jax.ShapeDtypeStruct((n, h_dim), F32),
    )(ntr, ctr, ptr, tcat, w3, pb)

    agg_fn = _make_agg(n, n_pad, h_dim, chunks_pad)
    for i, lyr in enumerate(params["layers"]):
        agg = agg_fn(h, src, dst, zeros_hbm)
        body = functools.partial(_layer_body, br=br, n_nodes=n,
                                 residual=(i > 0))
        h = pl.pallas_call(
            body,
            grid=(2, n_blocks),
            in_specs=[
                pl.BlockSpec((br, h_dim), lambda ph, j: (j, 0)),
                pl.BlockSpec((NC, br, h_dim), lambda ph, j: (0, j, 0)),
                pl.BlockSpec((h_dim, 2 * h_dim), lambda ph, j: (0, 0)),
                pl.BlockSpec((1, 2 * h_dim), lambda ph, j: (0, 0)),
                pl.BlockSpec((2 * h_dim, h_dim), lambda ph, j: (0, 0)),
                pl.BlockSpec((1, h_dim), lambda ph, j: (0, 0)),
                pl.BlockSpec((1, h_dim), lambda ph, j: (0, 0)),
                pl.BlockSpec((1, h_dim), lambda ph, j: (0, 0)),
            ],
            out_specs=pl.BlockSpec((br, h_dim), lambda ph, j: (j, 0)),
            out_shape=jax.ShapeDtypeStruct((n, h_dim), F32),
            scratch_shapes=[
                pltpu.VMEM((n, h_dim), F32),
                pltpu.VMEM((8, h_dim), F32),
            ],
        )(h, agg, lyr["W1"], lyr["b1"].reshape(1, -1),
          lyr["W2"], lyr["b2"].reshape(1, -1),
          lyr["gamma"].reshape(1, -1), lyr["beta"].reshape(1, -1))

    out = pl.pallas_call(
        functools.partial(_pool_body, br=br, n_graphs=n_graphs),
        grid=(n_blocks,),
        in_specs=[pl.BlockSpec((br, h_dim), lambda j: (j, 0)),
                  pl.BlockSpec((1, 1, br), lambda j: (j, 0, 0)),
                  pl.BlockSpec((1, br, 1), lambda j: (j, 0, 0))],
        out_specs=pl.BlockSpec((n_graphs, 3 * h_dim), lambda j: (0, 0)),
        out_shape=jax.ShapeDtypeStruct((n_graphs, 3 * h_dim), F32),
        scratch_shapes=[
            pltpu.VMEM((n_graphs, h_dim), F32),
            pltpu.VMEM((n_graphs, h_dim), F32),
            pltpu.VMEM((n_graphs, h_dim), F32),
        ],
    )(h, batch_r, batch_c)
    return out


# speedup vs baseline: 1.4942x; 1.4942x over previous
import functools

import jax
import jax.numpy as jnp
from jax import lax
from jax.experimental import pallas as pl
from jax.experimental.pallas import tpu as pltpu
from jax.experimental.pallas import tpu_sc as plsc

F32 = jnp.float32
I32 = jnp.int32

NC = 2
NS = 16
CH = 128
BN_EPS = 1e-5


GROUPS = 1


@functools.lru_cache(maxsize=None)
def _make_agg(n_nodes, n_pad, h_dim, chunks_pad):
    grp = chunks_pad // GROUPS
    rows_per_tile = n_pad // NS
    mesh = plsc.VectorSubcoreMesh(core_axis_name="c", subcore_axis_name="s")

    def body(h_hbm, src_hbm, dst_hbm, zeros_hbm, out_hbm,
             acc, sidx, didx, rows, *sems):
        c = lax.axis_index("c")
        s = lax.axis_index("s")
        lo = s * rows_per_tile
        pltpu.sync_copy(zeros_hbm.at[pl.ds(lo, rows_per_tile)],
                        acc.at[pl.ds(lo, rows_per_tile)])
        plsc.subcore_barrier()

        for g in range(GROUPS):
            pltpu.sync_copy(src_hbm.at[c, s, g], sidx)
            pltpu.sync_copy(dst_hbm.at[c, s, g], didx)
            @pl.loop(0, grp)
            def _(j):
                ds = pl.ds(j * CH, CH)
                pltpu.async_copy(h_hbm.at[sidx.at[ds]], rows, sems[0]).wait()
                pltpu.sync_copy(rows, acc.at[didx.at[ds]], add=True)

        plsc.subcore_barrier()
        pltpu.sync_copy(acc.at[pl.ds(lo, rows_per_tile)],
                        out_hbm.at[c, pl.ds(lo, rows_per_tile)])

    return pl.kernel(
        body,
        out_type=jax.ShapeDtypeStruct((NC, n_pad, h_dim), F32),
        mesh=mesh,
        scratch_types=[
            pltpu.VMEM_SHARED((n_pad, h_dim), F32),
            pltpu.VMEM((grp * CH,), I32),
            pltpu.VMEM((grp * CH,), I32),
            pltpu.VMEM((CH, h_dim), F32),
            pltpu.SemaphoreType.DMA,
        ],
    )


def _embed_body(ntr, ctr, ptr, tcat, w3, pb, out):
    br = out.shape[0]
    nt = ntr[0]
    ct = ctr[0]
    pt = ptr[0]
    i24 = lax.broadcasted_iota(I32, (br, 24), 1)
    oh = ((i24 == jnp.clip(nt, 0, 7))
          | (i24 == (jnp.clip(ct, 0, 7) + 8))
          | (i24 == (jnp.clip(pt, 0, 7) + 16))).astype(F32)
    t = tcat[...]
    w = w3[...]
    tp = jnp.concatenate(
        [jnp.dot(t[0:8], w[0], preferred_element_type=F32),
         jnp.dot(t[8:16], w[1], preferred_element_type=F32),
         jnp.dot(t[16:24], w[2], preferred_element_type=F32)], axis=0)
    out[...] = jnp.dot(oh, tp, preferred_element_type=F32) + pb[...]


def _layer_body(h_ref, agg_ref, w1, b1, w2, b2, gam, bet, out_ref,
                z2_s, st_s, *, br, n_nodes, residual):
    ph = pl.program_id(0)
    j = pl.program_id(1)

    @pl.when(ph == 0)
    def _():
        @pl.when(j == 0)
        def _():
            st_s[...] = jnp.zeros_like(st_s)
        z = h_ref[...] + agg_ref[0] + agg_ref[1]
        t = jnp.maximum(jnp.dot(z, w1[...], preferred_element_type=F32)
                        + b1[...], 0.0)
        z2 = jnp.dot(t, w2[...], preferred_element_type=F32) + b2[...]
        z2_s[pl.ds(j * br, br), :] = z2
        st_s[0:1, :] += jnp.sum(z2, axis=0, keepdims=True)
        st_s[1:2, :] += jnp.sum(z2 * z2, axis=0, keepdims=True)

    @pl.when(ph == 1)
    def _():
        nf = jnp.float32(n_nodes)
        mu = st_s[0:1, :] / nf
        var = st_s[1:2, :] / nf - mu * mu
        inv = 1.0 / jnp.sqrt(var + BN_EPS)
        z2 = z2_s[pl.ds(j * br, br), :]
        y = (z2 - mu) * inv * gam[...] + bet[...]
        y = jnp.maximum(y, 0.0)
        if residual:
            y = y + h_ref[...]
        out_ref[...] = y


def _pool_body(h_ref, brow_ref, bcol_ref, out_ref, sum_s, max_s, cnt_s,
               *, br, n_graphs):
    j = pl.program_id(0)

    @pl.when(j == 0)
    def _():
        sum_s[...] = jnp.zeros_like(sum_s)
        cnt_s[...] = jnp.zeros_like(cnt_s)
        max_s[...] = jnp.full_like(max_s, -jnp.inf)

    b_row = brow_ref[0]
    b_col = bcol_ref[0]
    h = h_ref[...]
    oht = (lax.broadcasted_iota(I32, (n_graphs, br), 0)
           == b_row).astype(F32)
    sum_s[...] += jnp.dot(oht, h, preferred_element_type=F32)
    cnt_s[...] += jnp.broadcast_to(jnp.sum(oht, axis=1, keepdims=True),
                                   cnt_s.shape)
    neg = jnp.float32(-jnp.inf)
    for g in range(n_graphs):
        m = jnp.max(jnp.where(b_col == g, h, neg), axis=0, keepdims=True)
        max_s[g:g + 1, :] = jnp.maximum(max_s[g:g + 1, :], m)

    cnt = jnp.maximum(cnt_s[...], 1.0)
    out_ref[...] = jnp.concatenate(
        [sum_s[...] / cnt, max_s[...], sum_s[...]], axis=1)


def kernel(x, edge_index, batch, params):
    n = x.shape[0]
    h_dim = params["proj_W"].shape[1]
    n_graphs = 16
    br = 1000
    n_blocks = n // br
    n_pad = ((n + 1 + NS * 8 - 1) // (NS * 8)) * (NS * 8)

    e = edge_index.shape[1]
    chunks_pad = -(-(-(-e // (NC * NS * CH))) // GROUPS) * GROUPS
    e_pad = NC * NS * chunks_pad * CH

    pad_dst = n + (jnp.arange(e_pad - e, dtype=I32) % (n_pad - n))
    src = jnp.concatenate(
        [edge_index[0], jnp.zeros((e_pad - e,), I32)]).reshape(
            NC, NS, GROUPS, (chunks_pad // GROUPS) * CH)
    dst = jnp.concatenate(
        [edge_index[1], pad_dst]).reshape(
            NC, NS, GROUPS, (chunks_pad // GROUPS) * CH)
    zeros_hbm = jnp.zeros((n_pad, h_dim), F32)

    tcat = jnp.zeros((24, h_dim), F32)
    tcat = tcat.at[0:5].set(params["node_emb"])
    tcat = tcat.at[8:14].set(params["comp_emb"])
    tcat = tcat.at[16:22].set(params["pin_emb"])
    w3 = params["proj_W"].reshape(3, h_dim, h_dim)
    pb = params["proj_b"].reshape(1, h_dim)

    xt = x.astype(I32).T.reshape(3, n_blocks, br, 1)
    ntr, ctr, ptr = xt[0], xt[1], xt[2]
    batch_r = batch.astype(I32).reshape(n_blocks, 1, br)
    batch_c = batch.astype(I32).reshape(n_blocks, br, 1)

    idx_spec = pl.BlockSpec((1, br, 1), lambda j: (j, 0, 0))
    full = lambda shp: pl.BlockSpec(shp, lambda j: tuple(0 for _ in shp))
    h = pl.pallas_call(
        _embed_body,
        grid=(n_blocks,),
        in_specs=[idx_spec, idx_spec, idx_spec,
                  full((24, h_dim)), full((3, h_dim, h_dim)),
                  full((1, h_dim))],
        out_specs=pl.BlockSpec((br, h_dim), lambda j: (j, 0)),
        out_shape=jax.ShapeDtypeStruct((n, h_dim), F32),
    )(ntr, ctr, ptr, tcat, w3, pb)

    agg_fn = _make_agg(n, n_pad, h_dim, chunks_pad)
    for i, lyr in enumerate(params["layers"]):
        agg = agg_fn(h, src, dst, zeros_hbm)
        body = functools.partial(_layer_body, br=br, n_nodes=n,
                                 residual=(i > 0))
        h = pl.pallas_call(
            body,
            grid=(2, n_blocks),
            in_specs=[
                pl.BlockSpec((br, h_dim), lambda ph, j: (j, 0)),
                pl.BlockSpec((NC, br, h_dim), lambda ph, j: (0, j, 0)),
                pl.BlockSpec((h_dim, 2 * h_dim), lambda ph, j: (0, 0)),
                pl.BlockSpec((1, 2 * h_dim), lambda ph, j: (0, 0)),
                pl.BlockSpec((2 * h_dim, h_dim), lambda ph, j: (0, 0)),
                pl.BlockSpec((1, h_dim), lambda ph, j: (0, 0)),
                pl.BlockSpec((1, h_dim), lambda ph, j: (0, 0)),
                pl.BlockSpec((1, h_dim), lambda ph, j: (0, 0)),
            ],
            out_specs=pl.BlockSpec((br, h_dim), lambda ph, j: (j, 0)),
            out_shape=jax.ShapeDtypeStruct((n, h_dim), F32),
            scratch_shapes=[
                pltpu.VMEM((n, h_dim), F32),
                pltpu.VMEM((8, h_dim), F32),
            ],
        )(h, agg, lyr["W1"], lyr["b1"].reshape(1, -1),
          lyr["W2"], lyr["b2"].reshape(1, -1),
          lyr["gamma"].reshape(1, -1), lyr["beta"].reshape(1, -1))

    out = pl.pallas_call(
        functools.partial(_pool_body, br=br, n_graphs=n_graphs),
        grid=(n_blocks,),
        in_specs=[pl.BlockSpec((br, h_dim), lambda j: (j, 0)),
                  pl.BlockSpec((1, 1, br), lambda j: (j, 0, 0)),
                  pl.BlockSpec((1, br, 1), lambda j: (j, 0, 0))],
        out_specs=pl.BlockSpec((n_graphs, 3 * h_dim), lambda j: (0, 0)),
        out_shape=jax.ShapeDtypeStruct((n_graphs, 3 * h_dim), F32),
        scratch_shapes=[
            pltpu.VMEM((n_graphs, h_dim), F32),
            pltpu.VMEM((n_graphs, h_dim), F32),
            pltpu.VMEM((n_graphs, h_dim), F32),
        ],
    )(h, batch_r, batch_c)
    return out
